# baseline (speedup 1.0000x reference)
import functools

import jax
import jax.numpy as jnp
from jax import lax
from jax.experimental import pallas as pl
from jax.experimental.pallas import tpu as pltpu
from jax.experimental.pallas import tpu_sc as plsc

NUM_EMB = 1_000_000
DIM = 32
B = 16384
S = 50
SP = 64
NC, NS = 2, 16
NW = NC * NS
B_PER_W = B // NW
CHUNK = 16
NCHUNK = B_PER_W // CHUNK
NBUF = 2


def _gather_body(table_hbm, idx_hbm, out_hbm, idx_v, rows_v, gsems, osems):
    wid = lax.axis_index("s") * NC + lax.axis_index("c")
    pltpu.sync_copy(idx_hbm.at[wid], idx_v)
    base_b = wid * B_PER_W

    def fire_chunk(c, buf):
        for k in range(CHUNK):
            pltpu.async_copy(
                table_hbm.at[idx_v.at[c * CHUNK + k]],
                rows_v.at[buf].at[pl.ds(k * SP, S)],
                gsems.at[buf],
            )

    def drain_chunk(buf):
        for _ in range(CHUNK):
            pltpu.make_async_copy(
                table_hbm.at[idx_v.at[0]],
                rows_v.at[buf].at[pl.ds(0, S)],
                gsems.at[buf],
            ).wait()

    for buf in range(NBUF):
        fire_chunk(buf, buf)

    def step(cc, carry):
        for buf in range(NBUF):
            c = cc * NBUF + buf
            drain_chunk(buf)
            pltpu.async_copy(
                rows_v.at[buf],
                out_hbm.at[pl.ds((base_b + c * CHUNK) * SP, CHUNK * SP)],
                osems.at[buf],
            )

            @pl.when(cc < NCHUNK // NBUF - 1)
            def _():
                pltpu.make_async_copy(
                    rows_v.at[buf],
                    out_hbm.at[pl.ds(base_b * SP, CHUNK * SP)],
                    osems.at[buf],
                ).wait()
                fire_chunk((cc + 1) * NBUF + buf, buf)

        return carry

    lax.fori_loop(0, NCHUNK // NBUF, step, 0)
    for buf in range(NBUF):
        pltpu.make_async_copy(
            rows_v.at[buf],
            out_hbm.at[pl.ds(base_b * SP, CHUNK * SP)],
            osems.at[buf],
        ).wait()


def _sc_gather(embedding_matrix, idx):
    run = pl.kernel(
        _gather_body,
        out_type=jax.ShapeDtypeStruct((B * SP, DIM), jnp.float32),
        mesh=plsc.VectorSubcoreMesh(core_axis_name="c", subcore_axis_name="s"),
        scratch_types=[
            pltpu.VMEM((B_PER_W, S), jnp.int32),
            pltpu.VMEM((NBUF, CHUNK * SP, DIM), jnp.float32),
            pltpu.SemaphoreType.DMA((NBUF,)),
            pltpu.SemaphoreType.DMA((NBUF,)),
        ],
        compiler_params=pltpu.CompilerParams(use_tc_tiling_on_sc=False),
    )
    return run(embedding_matrix, idx)


TCB = 4096
TGRID = (NUM_EMB + TCB - 1) // TCB


def _detile_body(t_ref, o_ref):
    x = t_ref[...]
    y = jnp.transpose(x)
    y3 = jnp.reshape(y, (TCB // 4, 4, DIM))
    o_ref[...] = jnp.concatenate(
        [y3[:, j, :] for j in range(4)], axis=1)


def _tc_detile(table_t):
    return pl.pallas_call(
        _detile_body,
        grid=(TGRID,),
        in_specs=[pl.BlockSpec((DIM, TCB), lambda i: (0, i))],
        out_specs=pl.BlockSpec((TCB * DIM // 128, 128), lambda i: (i, 0)),
        out_shape=jax.ShapeDtypeStruct((NUM_EMB * DIM // 128, 128), jnp.float32),
    )(table_t)


BB = 128
RPAD = SP * DIM
R = S * DIM


def _tp_body(a_ref, o_ref):
    x = a_ref[...]
    x2 = jnp.reshape(x, (BB, RPAD))
    y = jnp.transpose(x2)
    o_ref[...] = jnp.reshape(y[:R, :], (S, DIM, BB))


def _tc_transpose(g3):
    return pl.pallas_call(
        _tp_body,
        grid=(B // BB,),
        in_specs=[pl.BlockSpec((BB, RPAD // 128, 128), lambda i: (i, 0, 0))],
        out_specs=pl.BlockSpec((S, DIM, BB), lambda i: (0, 0, i)),
        out_shape=jax.ShapeDtypeStruct((S, DIM, B), jnp.float32),
    )(g3)


@functools.partial(jax.jit, static_argnums=())
def kernel(token_ids, embedding_matrix):
    idx = jnp.reshape(token_ids.astype(jnp.int32), (NW, B_PER_W, S))
    table_lin = jnp.reshape(_tc_detile(embedding_matrix.T), (NUM_EMB, DIM))
    g = _sc_gather(table_lin, idx)
    g3 = jnp.reshape(g, (B, RPAD // 128, 128))
    o = _tc_transpose(g3)
    return jnp.transpose(o, (2, 0, 1))


# speedup vs baseline: 1.2171x; 1.2171x over previous
import functools

import jax
import jax.numpy as jnp
from jax import lax
from jax.experimental import pallas as pl
from jax.experimental.pallas import tpu as pltpu
from jax.experimental.pallas import tpu_sc as plsc

NUM_EMB = 1_000_000
DIM = 32
B = 16384
S = 50
SP = 64
NC, NS = 2, 16
NW = NC * NS
B_PER_W = B // NW
CHUNK = 16
NCHUNK = B_PER_W // CHUNK
NBUF = 2


def _gather_body(table_hbm, idx_hbm, out_hbm, idx_v, rows_v, gsems, osems):
    wid = lax.axis_index("s") * NC + lax.axis_index("c")
    pltpu.sync_copy(idx_hbm.at[wid], idx_v)
    base_b = wid * B_PER_W

    def fire_chunk(c, buf):
        for k in range(CHUNK):
            pltpu.async_copy(
                table_hbm.at[idx_v.at[c * CHUNK + k]],
                rows_v.at[buf].at[pl.ds(k * SP, S)],
                gsems.at[buf],
            )

    def drain_chunk(buf):
        for _ in range(CHUNK):
            pltpu.make_async_copy(
                table_hbm.at[idx_v.at[0]],
                rows_v.at[buf].at[pl.ds(0, S)],
                gsems.at[buf],
            ).wait()

    for buf in range(NBUF):
        fire_chunk(buf, buf)

    def step(cc, carry):
        for buf in range(NBUF):
            c = cc * NBUF + buf
            drain_chunk(buf)
            pltpu.async_copy(
                rows_v.at[buf],
                out_hbm.at[pl.ds((base_b + c * CHUNK) * SP, CHUNK * SP)],
                osems.at[buf],
            )

            @pl.when(cc < NCHUNK // NBUF - 1)
            def _():
                pltpu.make_async_copy(
                    rows_v.at[buf],
                    out_hbm.at[pl.ds(base_b * SP, CHUNK * SP)],
                    osems.at[buf],
                ).wait()
                fire_chunk((cc + 1) * NBUF + buf, buf)

        return carry

    lax.fori_loop(0, NCHUNK // NBUF, step, 0)
    for buf in range(NBUF):
        pltpu.make_async_copy(
            rows_v.at[buf],
            out_hbm.at[pl.ds(base_b * SP, CHUNK * SP)],
            osems.at[buf],
        ).wait()


def _sc_gather(embedding_matrix, idx):
    run = pl.kernel(
        _gather_body,
        out_type=jax.ShapeDtypeStruct((B * SP, DIM), jnp.float32),
        mesh=plsc.VectorSubcoreMesh(core_axis_name="c", subcore_axis_name="s"),
        scratch_types=[
            pltpu.VMEM((B_PER_W, S), jnp.int32),
            pltpu.VMEM((NBUF, CHUNK * SP, DIM), jnp.float32),
            pltpu.SemaphoreType.DMA((NBUF,)),
            pltpu.SemaphoreType.DMA((NBUF,)),
        ],
        compiler_params=pltpu.CompilerParams(use_tc_tiling_on_sc=False),
    )
    return run(embedding_matrix, idx)


TCB = 8192
TGRID = (NUM_EMB + TCB - 1) // TCB


def _detile_body(t_ref, o_ref):
    x = t_ref[...]
    y = jnp.transpose(x)
    y3 = jnp.reshape(y, (TCB // 4, 4, DIM))
    qi = lax.broadcasted_iota(jnp.int32, (DIM, 128), 1)
    di = lax.broadcasted_iota(jnp.int32, (DIM, 128), 0)
    acc = jnp.zeros((TCB // 4, 128), jnp.float32)
    for j in range(4):
        sel = jnp.where(qi == DIM * j + di, 1.0, 0.0).astype(jnp.float32)
        acc = acc + jnp.dot(y3[:, j, :], sel,
                            preferred_element_type=jnp.float32)
    o_ref[...] = acc


def _tc_detile(table_t):
    return pl.pallas_call(
        _detile_body,
        grid=(TGRID,),
        in_specs=[pl.BlockSpec((DIM, TCB), lambda i: (0, i))],
        out_specs=pl.BlockSpec((TCB * DIM // 128, 128), lambda i: (i, 0)),
        out_shape=jax.ShapeDtypeStruct((NUM_EMB * DIM // 128, 128), jnp.float32),
    )(table_t)


BB = 256
RPAD = SP * DIM
R = S * DIM


def _tp_body(a_ref, o_ref):
    x = a_ref[...]
    x2 = jnp.reshape(x, (BB, RPAD))
    y = jnp.transpose(x2)
    o_ref[...] = jnp.reshape(y[:R, :], (S, DIM, BB))


def _tc_transpose(g3):
    return pl.pallas_call(
        _tp_body,
        grid=(B // BB,),
        in_specs=[pl.BlockSpec((BB, RPAD // 128, 128), lambda i: (i, 0, 0))],
        out_specs=pl.BlockSpec((S, DIM, BB), lambda i: (0, 0, i)),
        out_shape=jax.ShapeDtypeStruct((S, DIM, B), jnp.float32),
    )(g3)


@functools.partial(jax.jit, static_argnums=())
def kernel(token_ids, embedding_matrix):
    idx = jnp.reshape(token_ids.astype(jnp.int32), (NW, B_PER_W, S))
    table_lin = jnp.reshape(_tc_detile(embedding_matrix.T), (NUM_EMB, DIM))
    g = _sc_gather(table_lin, idx)
    g3 = jnp.reshape(g, (B, RPAD // 128, 128))
    o = _tc_transpose(g3)
    return jnp.transpose(o, (2, 0, 1))


# speedup vs baseline: 1.2203x; 1.0026x over previous
import functools

import jax
import jax.numpy as jnp
from jax import lax
from jax.experimental import pallas as pl
from jax.experimental.pallas import tpu as pltpu
from jax.experimental.pallas import tpu_sc as plsc

NUM_EMB = 1_000_000
DIM = 32
B = 16384
S = 50
SP = 64
NC, NS = 2, 16
NW = NC * NS
B_PER_W = B // NW
CHUNK = 16
NCHUNK = B_PER_W // CHUNK
NBUF = 2


def _gather_body(table_hbm, idx_hbm, out_hbm, idx_v, rows_v, gsems, osems):
    wid = lax.axis_index("s") * NC + lax.axis_index("c")
    pltpu.sync_copy(idx_hbm.at[wid], idx_v)
    base_b = wid * B_PER_W

    def fire_chunk(c, buf):
        for k in range(CHUNK):
            pltpu.async_copy(
                table_hbm.at[idx_v.at[c * CHUNK + k]],
                rows_v.at[buf].at[pl.ds(k * SP, S)],
                gsems.at[buf],
            )

    def drain_chunk(buf):
        for _ in range(CHUNK):
            pltpu.make_async_copy(
                table_hbm.at[idx_v.at[0]],
                rows_v.at[buf].at[pl.ds(0, S)],
                gsems.at[buf],
            ).wait()

    for buf in range(NBUF):
        fire_chunk(buf, buf)

    def step(cc, carry):
        for buf in range(NBUF):
            c = cc * NBUF + buf
            drain_chunk(buf)
            pltpu.async_copy(
                rows_v.at[buf],
                out_hbm.at[pl.ds((base_b + c * CHUNK) * SP, CHUNK * SP)],
                osems.at[buf],
            )

            @pl.when(cc < NCHUNK // NBUF - 1)
            def _():
                pltpu.make_async_copy(
                    rows_v.at[buf],
                    out_hbm.at[pl.ds(base_b * SP, CHUNK * SP)],
                    osems.at[buf],
                ).wait()
                fire_chunk((cc + 1) * NBUF + buf, buf)

        return carry

    lax.fori_loop(0, NCHUNK // NBUF, step, 0)
    for buf in range(NBUF):
        pltpu.make_async_copy(
            rows_v.at[buf],
            out_hbm.at[pl.ds(base_b * SP, CHUNK * SP)],
            osems.at[buf],
        ).wait()


def _sc_gather(embedding_matrix, idx):
    run = pl.kernel(
        _gather_body,
        out_type=jax.ShapeDtypeStruct((B * SP, DIM), jnp.float32),
        mesh=plsc.VectorSubcoreMesh(core_axis_name="c", subcore_axis_name="s"),
        scratch_types=[
            pltpu.VMEM((B_PER_W, S), jnp.int32),
            pltpu.VMEM((NBUF, CHUNK * SP, DIM), jnp.float32),
            pltpu.SemaphoreType.DMA((NBUF,)),
            pltpu.SemaphoreType.DMA((NBUF,)),
        ],
        compiler_params=pltpu.CompilerParams(use_tc_tiling_on_sc=False),
    )
    return run(embedding_matrix, idx)


TCB = 16384
TGRID = (NUM_EMB + TCB - 1) // TCB


def _detile_body(t_ref, o_ref):
    x = t_ref[...]
    y = jnp.transpose(x)
    y3 = jnp.reshape(y, (TCB // 4, 4, DIM))
    qi = lax.broadcasted_iota(jnp.int32, (DIM, 128), 1)
    di = lax.broadcasted_iota(jnp.int32, (DIM, 128), 0)
    acc = jnp.zeros((TCB // 4, 128), jnp.float32)
    for j in range(4):
        sel = jnp.where(qi == DIM * j + di, 1.0, 0.0).astype(jnp.float32)
        acc = acc + jnp.dot(y3[:, j, :], sel,
                            preferred_element_type=jnp.float32)
    o_ref[...] = acc


def _tc_detile(table_t):
    return pl.pallas_call(
        _detile_body,
        grid=(TGRID,),
        in_specs=[pl.BlockSpec((DIM, TCB), lambda i: (0, i))],
        out_specs=pl.BlockSpec((TCB * DIM // 128, 128), lambda i: (i, 0)),
        out_shape=jax.ShapeDtypeStruct((NUM_EMB * DIM // 128, 128), jnp.float32),
    )(table_t)


BB = 256
RPAD = SP * DIM
R = S * DIM


def _tp_body(a_ref, o_ref):
    x = a_ref[...]
    x2 = jnp.reshape(x, (BB, RPAD))
    y = jnp.transpose(x2)
    o_ref[...] = jnp.reshape(y[:R, :], (S, DIM, BB))


def _tc_transpose(g3):
    return pl.pallas_call(
        _tp_body,
        grid=(B // BB,),
        in_specs=[pl.BlockSpec((BB, RPAD // 128, 128), lambda i: (i, 0, 0))],
        out_specs=pl.BlockSpec((S, DIM, BB), lambda i: (0, 0, i)),
        out_shape=jax.ShapeDtypeStruct((S, DIM, B), jnp.float32),
    )(g3)


@functools.partial(jax.jit, static_argnums=())
def kernel(token_ids, embedding_matrix):
    idx = jnp.reshape(token_ids.astype(jnp.int32), (NW, B_PER_W, S))
    table_lin = jnp.reshape(_tc_detile(embedding_matrix.T), (NUM_EMB, DIM))
    g = _sc_gather(table_lin, idx)
    g3 = jnp.reshape(g, (B, RPAD // 128, 128))
    o = _tc_transpose(g3)
    return jnp.transpose(o, (2, 0, 1))


# speedup vs baseline: 1.2598x; 1.0324x over previous
import functools

import jax
import jax.numpy as jnp
from jax import lax
from jax.experimental import pallas as pl
from jax.experimental.pallas import tpu as pltpu
from jax.experimental.pallas import tpu_sc as plsc

NUM_EMB = 1_000_000
DIM = 32
B = 16384
S = 50
SP = 64
NC, NS = 2, 16
NW = NC * NS
B_PER_W = B // NW
CHUNK = 16
NCHUNK = B_PER_W // CHUNK
NBUF = 2


def _gather_body(table_hbm, idx_hbm, out_hbm, idx_v, rows_v, gsems, osems):
    wid = lax.axis_index("s") * NC + lax.axis_index("c")
    pltpu.sync_copy(idx_hbm.at[wid], idx_v)
    base_b = wid * B_PER_W

    def fire_chunk(c, buf):
        for k in range(CHUNK):
            pltpu.async_copy(
                table_hbm.at[idx_v.at[c * CHUNK + k]],
                rows_v.at[buf].at[pl.ds(k * SP, S)],
                gsems.at[buf],
            )

    def drain_chunk(buf):
        for _ in range(CHUNK):
            pltpu.make_async_copy(
                table_hbm.at[idx_v.at[0]],
                rows_v.at[buf].at[pl.ds(0, S)],
                gsems.at[buf],
            ).wait()

    for buf in range(NBUF):
        fire_chunk(buf, buf)

    def step(cc, carry):
        for buf in range(NBUF):
            c = cc * NBUF + buf
            drain_chunk(buf)
            pltpu.async_copy(
                rows_v.at[buf],
                out_hbm.at[pl.ds((base_b + c * CHUNK) * SP, CHUNK * SP)],
                osems.at[buf],
            )

            @pl.when(cc < NCHUNK // NBUF - 1)
            def _():
                pltpu.make_async_copy(
                    rows_v.at[buf],
                    out_hbm.at[pl.ds(base_b * SP, CHUNK * SP)],
                    osems.at[buf],
                ).wait()
                fire_chunk((cc + 1) * NBUF + buf, buf)

        return carry

    lax.fori_loop(0, NCHUNK // NBUF, step, 0)
    for buf in range(NBUF):
        pltpu.make_async_copy(
            rows_v.at[buf],
            out_hbm.at[pl.ds(base_b * SP, CHUNK * SP)],
            osems.at[buf],
        ).wait()


def _sc_gather(embedding_matrix, idx):
    run = pl.kernel(
        _gather_body,
        out_type=jax.ShapeDtypeStruct((B * SP, DIM), jnp.float32),
        mesh=plsc.VectorSubcoreMesh(core_axis_name="c", subcore_axis_name="s"),
        scratch_types=[
            pltpu.VMEM((B_PER_W, S), jnp.int32),
            pltpu.VMEM((NBUF, CHUNK * SP, DIM), jnp.float32),
            pltpu.SemaphoreType.DMA((NBUF,)),
            pltpu.SemaphoreType.DMA((NBUF,)),
        ],
        compiler_params=pltpu.CompilerParams(use_tc_tiling_on_sc=False),
    )
    return run(embedding_matrix, idx)


TCB = 16384
TGRID = (NUM_EMB + TCB - 1) // TCB


def _detile_body(t_ref, o_ref):
    x = t_ref[...]
    y = jnp.transpose(x)
    y3 = jnp.reshape(y, (TCB // 4, 4, DIM))
    qi = lax.broadcasted_iota(jnp.int32, (DIM, 128), 1)
    di = lax.broadcasted_iota(jnp.int32, (DIM, 128), 0)
    acc = jnp.zeros((TCB // 4, 128), jnp.float32)
    for j in range(4):
        sel = jnp.where(qi == DIM * j + di, 1.0, 0.0).astype(jnp.float32)
        acc = acc + jnp.dot(y3[:, j, :], sel,
                            preferred_element_type=jnp.float32)
    o_ref[...] = acc


def _tc_detile(table_t):
    return pl.pallas_call(
        _detile_body,
        grid=(TGRID,),
        in_specs=[pl.BlockSpec((DIM, TCB), lambda i: (0, i))],
        out_specs=pl.BlockSpec((TCB * DIM // 128, 128), lambda i: (i, 0)),
        out_shape=jax.ShapeDtypeStruct((NUM_EMB * DIM // 128, 128), jnp.float32),
    )(table_t)


BB = 512
RPAD = SP * DIM
R = S * DIM


def _tp_body(a_ref, o_ref):
    x = a_ref[...]
    x2 = jnp.reshape(x, (BB, RPAD))
    y = jnp.transpose(x2)
    o_ref[...] = jnp.reshape(y[:R, :], (S, DIM, BB))


def _tc_transpose(g3):
    return pl.pallas_call(
        _tp_body,
        grid=(B // BB,),
        in_specs=[pl.BlockSpec((BB, RPAD // 128, 128), lambda i: (i, 0, 0))],
        out_specs=pl.BlockSpec((S, DIM, BB), lambda i: (0, 0, i)),
        out_shape=jax.ShapeDtypeStruct((S, DIM, B), jnp.float32),
    )(g3)


@functools.partial(jax.jit, static_argnums=())
def kernel(token_ids, embedding_matrix):
    idx = jnp.reshape(token_ids.astype(jnp.int32), (NW, B_PER_W, S))
    table_lin = jnp.reshape(_tc_detile(embedding_matrix.T), (NUM_EMB, DIM))
    g = _sc_gather(table_lin, idx)
    g3 = jnp.reshape(g, (B, RPAD // 128, 128))
    o = _tc_transpose(g3)
    return jnp.transpose(o, (2, 0, 1))


# speedup vs baseline: 1.2708x; 1.0087x over previous
import functools

import jax
import jax.numpy as jnp
from jax import lax
from jax.experimental import pallas as pl
from jax.experimental.pallas import tpu as pltpu
from jax.experimental.pallas import tpu_sc as plsc

NUM_EMB = 1_000_000
DIM = 32
B = 16384
S = 50
SP = 64
NC, NS = 2, 16
NW = NC * NS
B_PER_W = B // NW
CHUNK = 16
NCHUNK = B_PER_W // CHUNK
NBUF = 2


def _gather_body(table_hbm, idx_hbm, out_hbm, idx_v, rows_v, gsems, osems):
    wid = lax.axis_index("s") * NC + lax.axis_index("c")
    pltpu.sync_copy(idx_hbm.at[wid], idx_v)
    base_b = wid * B_PER_W

    def fire_chunk(c, buf):
        for k in range(CHUNK):
            pltpu.async_copy(
                table_hbm.at[idx_v.at[c * CHUNK + k]],
                rows_v.at[buf].at[pl.ds(k * SP, S)],
                gsems.at[buf],
            )

    def drain_chunk(buf):
        for _ in range(CHUNK):
            pltpu.make_async_copy(
                table_hbm.at[idx_v.at[0]],
                rows_v.at[buf].at[pl.ds(0, S)],
                gsems.at[buf],
            ).wait()

    for buf in range(NBUF):
        fire_chunk(buf, buf)

    def step(cc, carry):
        for buf in range(NBUF):
            c = cc * NBUF + buf
            drain_chunk(buf)
            pltpu.async_copy(
                rows_v.at[buf],
                out_hbm.at[pl.ds((base_b + c * CHUNK) * SP, CHUNK * SP)],
                osems.at[buf],
            )

            @pl.when(cc < NCHUNK // NBUF - 1)
            def _():
                pltpu.make_async_copy(
                    rows_v.at[buf],
                    out_hbm.at[pl.ds(base_b * SP, CHUNK * SP)],
                    osems.at[buf],
                ).wait()
                fire_chunk((cc + 1) * NBUF + buf, buf)

        return carry

    lax.fori_loop(0, NCHUNK // NBUF, step, 0)
    for buf in range(NBUF):
        pltpu.make_async_copy(
            rows_v.at[buf],
            out_hbm.at[pl.ds(base_b * SP, CHUNK * SP)],
            osems.at[buf],
        ).wait()


def _sc_gather(embedding_matrix, idx):
    run = pl.kernel(
        _gather_body,
        out_type=jax.ShapeDtypeStruct((B * SP, DIM), jnp.float32),
        mesh=plsc.VectorSubcoreMesh(core_axis_name="c", subcore_axis_name="s"),
        scratch_types=[
            pltpu.VMEM((B_PER_W, S), jnp.int32),
            pltpu.VMEM((NBUF, CHUNK * SP, DIM), jnp.float32),
            pltpu.SemaphoreType.DMA((NBUF,)),
            pltpu.SemaphoreType.DMA((NBUF,)),
        ],
        compiler_params=pltpu.CompilerParams(use_tc_tiling_on_sc=False),
    )
    return run(embedding_matrix, idx)


TCB = 32768
TGRID = (NUM_EMB + TCB - 1) // TCB


def _detile_body(t_ref, o_ref):
    x = t_ref[...]
    y = jnp.transpose(x)
    y3 = jnp.reshape(y, (TCB // 4, 4, DIM))
    qi = lax.broadcasted_iota(jnp.int32, (DIM, 128), 1)
    di = lax.broadcasted_iota(jnp.int32, (DIM, 128), 0)
    acc = jnp.zeros((TCB // 4, 128), jnp.float32)
    for j in range(4):
        sel = jnp.where(qi == DIM * j + di, 1.0, 0.0).astype(jnp.float32)
        acc = acc + jnp.dot(y3[:, j, :], sel,
                            preferred_element_type=jnp.float32)
    o_ref[...] = acc


def _tc_detile(table_t):
    return pl.pallas_call(
        _detile_body,
        grid=(TGRID,),
        in_specs=[pl.BlockSpec((DIM, TCB), lambda i: (0, i))],
        out_specs=pl.BlockSpec((TCB * DIM // 128, 128), lambda i: (i, 0)),
        out_shape=jax.ShapeDtypeStruct((NUM_EMB * DIM // 128, 128), jnp.float32),
    )(table_t)


BB = 1024
RPAD = SP * DIM
R = S * DIM


def _tp_body(a_ref, o_ref):
    x = a_ref[...]
    x2 = jnp.reshape(x, (BB, RPAD))
    y = jnp.transpose(x2)
    o_ref[...] = jnp.reshape(y[:R, :], (S, DIM, BB))


def _tc_transpose(g3):
    return pl.pallas_call(
        _tp_body,
        grid=(B // BB,),
        in_specs=[pl.BlockSpec((BB, RPAD // 128, 128), lambda i: (i, 0, 0))],
        out_specs=pl.BlockSpec((S, DIM, BB), lambda i: (0, 0, i)),
        out_shape=jax.ShapeDtypeStruct((S, DIM, B), jnp.float32),
    )(g3)


@functools.partial(jax.jit, static_argnums=())
def kernel(token_ids, embedding_matrix):
    idx = jnp.reshape(token_ids.astype(jnp.int32), (NW, B_PER_W, S))
    table_lin = jnp.reshape(_tc_detile(embedding_matrix.T), (NUM_EMB, DIM))
    g = _sc_gather(table_lin, idx)
    g3 = jnp.reshape(g, (B, RPAD // 128, 128))
    o = _tc_transpose(g3)
    return jnp.transpose(o, (2, 0, 1))
